# baseline (speedup 1.0000x reference)
import functools

import jax
import jax.numpy as jnp
from jax import lax
from jax.experimental import pallas as pl
from jax.experimental.pallas import tpu as pltpu
from jax.experimental.pallas import tpu_sc as plsc

NU, NI, NB = 20000, 40000, 20000
D, H, W = 64, 32, 16
N_IL = NU + NI
N_BL = NU + NB
E_UI, E_UB, E_BI = 500_000, 300_000, 400_000

NT = 16
NC = 2
BATCH = 1024
ZR = 512

LEN_IL, LEN_BL, LEN_BI = 60032, 40064, 20096


def _ceil_to(x, m):
    return (x + m - 1) // m * m


E_IL_PAD = _ceil_to(2 * E_UI, 3 * NT * BATCH)
E_BL_PAD = _ceil_to(2 * E_UB, 3 * NT * BATCH)
E_BI_PAD = _ceil_to(E_BI, 3 * NT * BATCH)

DEG_E = 2 * E_UI + 2 * E_UB + E_BI
DEG_EPAD = _ceil_to(DEG_E, 3 * NC * NT * BATCH)
OFF_BL = LEN_IL
OFF_BI = LEN_IL + LEN_BL
N_DEG = LEN_IL + LEN_BL + LEN_BI
DUMMY_DEG = N_IL


def _mesh():
    return plsc.VectorSubcoreMesh(core_axis_name="c", subcore_axis_name="s")


@functools.cache
def _deg_kernel():
    half = DEG_EPAD // NC
    chunk = half // NT
    nb = chunk // BATCH
    nzz = N_DEG // NT
    n1, n2, n3 = LEN_IL // NT, LEN_BL // NT, LEN_BI // NT

    nb3 = nb // 3

    def body(didx, o_il, o_bl, o_bi, idxv0, idxv1, idxv2, onesv, zbuf, acc,
             isem0, isem1, isem2, ssem0, ssem1, ssem2):
        c = lax.axis_index("c")
        s = lax.axis_index("s")
        idxv = (idxv0, idxv1, idxv2)
        isem = (isem0, isem1, isem2)
        ssem = (ssem0, ssem1, ssem2)
        ones16 = jnp.ones((16,), jnp.float32)
        zero16 = jnp.zeros((16,), jnp.float32)

        def fill(i, _):
            zbuf[pl.ds(i * 16, 16)] = zero16
            return 0

        lax.fori_loop(0, 2048 // 16, fill, 0)

        def fill1(i, _):
            onesv[pl.ds(i * 16, 16)] = ones16
            return 0

        lax.fori_loop(0, BATCH // 16, fill1, 0)

        off = 0
        while off < nzz:
            sz = min(2048, nzz - off)
            pltpu.sync_copy(zbuf.at[pl.ds(0, sz)], acc.at[pl.ds(s * nzz + off, sz)])
            off += sz
        plsc.subcore_barrier()

        base = c * half + s * chunk

        def idx_start(b, k):
            pltpu.async_copy(didx.at[pl.ds(base + b * BATCH, BATCH)], idxv[k], isem[k])

        def idx_wait(b, k):
            pltpu.make_async_copy(
                didx.at[pl.ds(base + b * BATCH, BATCH)], idxv[k], isem[k]).wait()

        def scat_start(k):
            pltpu.async_copy(onesv, acc.at[idxv[k]], ssem[k], add=True)

        def scat_wait(k):
            pltpu.make_async_copy(onesv, acc.at[idxv[k]], ssem[k]).wait()

        def slot(b, k, do_c):
            k2 = (k + 1) % 3
            idx_wait(b, k)
            scat_start(k)
            if do_c:
                scat_wait(k2)
            idx_start(jnp.minimum(b + 1, nb - 1), k2)

        idx_start(0, 0)
        slot(0, 0, False)
        slot(1, 1, False)
        slot(2, 2, True)

        def outer(j, _):
            b = 3 * j
            slot(b, 0, True)
            slot(b + 1, 1, True)
            slot(b + 2, 2, True)
            return 0

        lax.fori_loop(1, nb3, outer, 0)
        scat_wait(1)
        scat_wait(2)
        idx_wait(nb - 1, 0)
        plsc.subcore_barrier()

        def drain(a_off, out, o_off, total):
            done = 0
            while done < total:
                sz = min(2048, total - done)
                pltpu.sync_copy(acc.at[pl.ds(a_off + done, sz)], zbuf.at[pl.ds(0, sz)])
                pltpu.sync_copy(zbuf.at[pl.ds(0, sz)], out.at[pl.ds(o_off + done, sz)])
                done += sz

        drain(s * n1, o_il, c * LEN_IL + s * n1, n1)
        drain(OFF_BL + s * n2, o_bl, c * LEN_BL + s * n2, n2)
        drain(OFF_BI + s * n3, o_bi, c * LEN_BI + s * n3, n3)

    return pl.kernel(
        body,
        out_type=[
            jax.ShapeDtypeStruct((NC * LEN_IL,), jnp.float32),
            jax.ShapeDtypeStruct((NC * LEN_BL,), jnp.float32),
            jax.ShapeDtypeStruct((NC * LEN_BI,), jnp.float32),
        ],
        mesh=_mesh(),
        compiler_params=pltpu.CompilerParams(use_tc_tiling_on_sc=False),
        scratch_types=[
            pltpu.VMEM((BATCH,), jnp.int32),
            pltpu.VMEM((BATCH,), jnp.int32),
            pltpu.VMEM((BATCH,), jnp.int32),
            pltpu.VMEM((BATCH,), jnp.float32),
            pltpu.VMEM((2048,), jnp.float32),
            pltpu.VMEM_SHARED((N_DEG,), jnp.float32),
            pltpu.SemaphoreType.DMA,
            pltpu.SemaphoreType.DMA,
            pltpu.SemaphoreType.DMA,
            pltpu.SemaphoreType.DMA,
            pltpu.SemaphoreType.DMA,
            pltpu.SemaphoreType.DMA,
        ],
    )


@functools.cache
def _spmm(n_pad, e_pad, pass_off):
    chunk = e_pad // NT
    nb = chunk // BATCH
    nb3 = nb // 3
    nro = n_pad // NT

    def body(table, cols4, rows, out,
             colv0, colv1, colv2, rowv0, rowv1, rowv2,
             gath0, gath1, gath2, ztile, acc,
             isem0, isem1, isem2, gsem0, gsem1, gsem2, ssem0, ssem1, ssem2):
        c = lax.axis_index("c")
        s = lax.axis_index("s")
        colv = (colv0, colv1, colv2)
        rowv = (rowv0, rowv1, rowv2)
        gath = (gath0, gath1, gath2)
        isem = (isem0, isem1, isem2)
        gsem = (gsem0, gsem1, gsem2)
        ssem = (ssem0, ssem1, ssem2)
        zv = jnp.zeros((16,), jnp.float32)

        def zfill(i, _):
            ztile[i, pl.ds(0, 16)] = zv
            return 0

        lax.fori_loop(0, ZR, zfill, 0)
        off = 0
        while off < nro:
            sz = min(ZR, nro - off)
            pltpu.sync_copy(ztile.at[pl.ds(0, sz)], acc.at[pl.ds(s * nro + off, sz)])
            off += sz
        plsc.subcore_barrier()

        base = s * chunk

        def idx_start(b, k):
            eb = base + b * BATCH
            pltpu.async_copy(cols4.at[pass_off + c, pl.ds(eb, BATCH)], colv[k], isem[k])
            pltpu.async_copy(rows.at[pl.ds(eb, BATCH)], rowv[k], isem[k])

        def idx_wait(b, k):
            eb = base + b * BATCH
            pltpu.make_async_copy(
                cols4.at[pass_off + c, pl.ds(eb, BATCH)], colv[k], isem[k]).wait()
            pltpu.make_async_copy(rows.at[pl.ds(eb, BATCH)], rowv[k], isem[k]).wait()

        def gath_start(k):
            pltpu.async_copy(table.at[colv[k]], gath[k], gsem[k])

        def gath_wait(k):
            pltpu.make_async_copy(table.at[colv[k]], gath[k], gsem[k]).wait()

        def scat_start(k):
            pltpu.async_copy(gath[k], acc.at[rowv[k]], ssem[k], add=True)

        def scat_wait(k):
            pltpu.make_async_copy(gath[k], acc.at[rowv[k]], ssem[k]).wait()

        def slot(b, k, do_c, do_e):
            k1 = (k + 2) % 3
            k2 = (k + 1) % 3
            idx_wait(b, k)
            gath_start(k)
            if do_c:
                gath_wait(k1)
                scat_start(k1)
            if do_e:
                scat_wait(k2)
            idx_start(jnp.minimum(b + 1, nb - 1), k2)

        idx_start(0, 0)
        slot(0, 0, False, False)
        slot(1, 1, True, False)
        slot(2, 2, True, True)

        def outer(j, _):
            b = 3 * j
            slot(b, 0, True, True)
            slot(b + 1, 1, True, True)
            slot(b + 2, 2, True, True)
            return 0

        lax.fori_loop(1, nb3, outer, 0)
        gath_wait(2)
        scat_start(2)
        scat_wait(1)
        scat_wait(2)
        idx_wait(nb - 1, 0)
        plsc.subcore_barrier()
        done = 0
        while done < nro:
            sz = min(ZR, nro - done)
            pltpu.sync_copy(acc.at[pl.ds(s * nro + done, sz)], ztile.at[pl.ds(0, sz)])
            pltpu.sync_copy(ztile.at[pl.ds(0, sz)], out.at[c, pl.ds(s * nro + done, sz)])
            done += sz

    return pl.kernel(
        body,
        out_type=jax.ShapeDtypeStruct((NC, n_pad, W), jnp.float32),
        mesh=_mesh(),
        compiler_params=pltpu.CompilerParams(use_tc_tiling_on_sc=False),
        scratch_types=(
            [pltpu.VMEM((BATCH,), jnp.int32)] * 6
            + [pltpu.VMEM((BATCH, W), jnp.float32)] * 3
            + [
                pltpu.VMEM((ZR, W), jnp.float32),
                pltpu.VMEM_SHARED((n_pad, W), jnp.float32),
            ]
            + [pltpu.SemaphoreType.DMA] * 9
        ),
    )


BR = 128


def _dinv_of(dref):
    d = dref[0] + dref[1]
    return 1.0 / (jnp.sqrt(d) + 1e-8)


def _quarters(oref, x):
    for q in range(4):
        oref[q] = x[:, q * W:(q + 1) * W]


def _tc_prescale(f0, deg):
    n = f0.shape[0]

    def body(fref, dref, oref):
        g = fref[...] * _dinv_of(dref)
        _quarters(oref, g)

    return pl.pallas_call(
        body,
        grid=(n // BR,),
        in_specs=[
            pl.BlockSpec((BR, D), lambda i: (i, 0)),
            pl.BlockSpec((2, BR, 1), lambda i: (0, i, 0)),
        ],
        out_specs=pl.BlockSpec((4, BR, W), lambda i: (0, i, 0)),
        out_shape=jax.ShapeDtypeStruct((4, n, W), jnp.float32),
    )(f0, deg)


def _tc_post(sa, sb, deg, prev, scale, want_g, want_ostk):
    n = prev.shape[0]
    out_shape = [jax.ShapeDtypeStruct((n, D), jnp.float32)]
    out_specs = [pl.BlockSpec((BR, D), lambda i: (i, 0))]
    if want_g or want_ostk:
        out_shape.append(jax.ShapeDtypeStruct((4, n, W), jnp.float32))
        out_specs.append(pl.BlockSpec((4, BR, W), lambda i: (0, i, 0)))

    def body(saref, sbref, dref, pref, oref, *rest):
        dinv = _dinv_of(dref)
        s2 = jnp.concatenate([saref[0], saref[1], sbref[0], sbref[1]], axis=1)
        f = s2 * dinv * scale
        nrm = jnp.sqrt(jnp.sum(f * f, axis=1, keepdims=True))
        a = f / (nrm + 1e-12)
        p = pref[...] + a
        oref[...] = p
        if want_g:
            _quarters(rest[0], f * dinv)
        elif want_ostk:
            _quarters(rest[0], p)

    return pl.pallas_call(
        body,
        grid=(n // BR,),
        in_specs=[
            pl.BlockSpec((2, BR, W), lambda i: (0, i, 0)),
            pl.BlockSpec((2, BR, W), lambda i: (0, i, 0)),
            pl.BlockSpec((2, BR, 1), lambda i: (0, i, 0)),
            pl.BlockSpec((BR, D), lambda i: (i, 0)),
        ],
        out_specs=out_specs,
        out_shape=out_shape,
    )(sa, sb, deg, prev)


def _tc_post_bi(sa, sb, deg):
    n = deg.shape[1]

    def body(saref, sbref, dref, oref):
        binv = 1.0 / (dref[0] + dref[1] + 1e-8)
        s2 = jnp.concatenate([saref[0], saref[1], sbref[0], sbref[1]], axis=1)
        oref[...] = s2 * binv

    return pl.pallas_call(
        body,
        grid=(n // BR,),
        in_specs=[
            pl.BlockSpec((2, BR, W), lambda i: (0, i, 0)),
            pl.BlockSpec((2, BR, W), lambda i: (0, i, 0)),
            pl.BlockSpec((2, BR, 1), lambda i: (0, i, 0)),
        ],
        out_specs=pl.BlockSpec((BR, D), lambda i: (i, 0)),
        out_shape=jax.ShapeDtypeStruct((n, D), jnp.float32),
    )(sa, sb, deg)


def _pad_i32(x, length, value):
    return jnp.concatenate([x, jnp.full((length - x.shape[0],), value, jnp.int32)])


def _spmm4(table4, cols4, rows, n_pad):
    tbl = table4.reshape(-1, W)
    e_pad = rows.shape[0]
    sa = _spmm(n_pad, e_pad, 0)(tbl, cols4, rows)
    sb = _spmm(n_pad, e_pad, 2)(tbl, cols4, rows)
    return sa, sb


def _propagate(f0, rows, cols4, deg, n_pad):
    g0 = _tc_prescale(f0, deg)
    sa1, sb1 = _spmm4(g0, cols4, rows, n_pad)
    p1, g1 = _tc_post(sa1, sb1, deg, f0, 0.5, True, False)
    sa2, sb2 = _spmm4(g1, cols4, rows, n_pad)
    return _tc_post(sa2, sb2, deg, p1, 1.0 / 3.0, False, True)


def _cols4_of(cols, n_pad):
    return jnp.stack([cols, cols + n_pad, cols + 2 * n_pad, cols + 3 * n_pad])


def kernel(users_feature, items_feature, bundles_feature, ui_edges, ub_edges, bi_edges):
    ur, uc = ui_edges[0], ui_edges[1]
    vr, vc = ub_edges[0], ub_edges[1]
    br_, bc_ = bi_edges[0], bi_edges[1]

    deg_idx = jnp.concatenate([
        ur, uc + NU,
        vr + OFF_BL, vc + (NU + OFF_BL),
        br_ + OFF_BI,
        jnp.full((DEG_EPAD - DEG_E,), DUMMY_DEG, jnp.int32),
    ])
    d_il, d_bl, d_bi = _deg_kernel()(deg_idx)
    deg_il = d_il.reshape(NC, LEN_IL, 1)
    deg_bl = d_bl.reshape(NC, LEN_BL, 1)
    deg_bi = d_bi.reshape(NC, LEN_BI, 1)

    zpad_il = jnp.zeros((LEN_IL - N_IL, D), jnp.float32)
    zpad_bl = jnp.zeros((LEN_BL - N_BL, D), jnp.float32)

    f0_il = jnp.concatenate([users_feature, items_feature, zpad_il], axis=0)
    rows_il = _pad_i32(jnp.concatenate([ur, uc + NU]), E_IL_PAD, N_IL)
    cols_il = _pad_i32(jnp.concatenate([uc + NU, ur]), E_IL_PAD, 0)
    il_out, il_stk = _propagate(f0_il, rows_il, _cols4_of(cols_il, LEN_IL), deg_il, LEN_IL)

    f0_bl = jnp.concatenate([users_feature, bundles_feature, zpad_bl], axis=0)
    rows_bl = _pad_i32(jnp.concatenate([vr, vc + NU]), E_BL_PAD, N_BL)
    cols_bl = _pad_i32(jnp.concatenate([vc + NU, vr]), E_BL_PAD, 0)
    bl_out, _ = _propagate(f0_bl, rows_bl, _cols4_of(cols_bl, LEN_BL), deg_bl, LEN_BL)

    rows_bi = _pad_i32(br_, E_BI_PAD, NB)
    cols_bi = _pad_i32(bc_ + NU, E_BI_PAD, 0)
    sa, sb = _spmm4(il_stk, _cols4_of(cols_bi, LEN_IL), rows_bi, LEN_BI)
    il_b = _tc_post_bi(sa, sb, deg_bi)

    users_rep = jnp.concatenate([il_out[:NU], bl_out[:NU]], axis=1)
    bundles_rep = jnp.concatenate([il_b[:NB], bl_out[NU:N_BL]], axis=1)
    return users_rep, bundles_rep


# speedup vs baseline: 1.1033x; 1.1033x over previous
import functools

import jax
import jax.numpy as jnp
from jax import lax
from jax.experimental import pallas as pl
from jax.experimental.pallas import tpu as pltpu
from jax.experimental.pallas import tpu_sc as plsc

NU, NI, NB = 20000, 40000, 20000
D, H, W = 64, 32, 16
N_IL = NU + NI
N_BL = NU + NB
E_UI, E_UB, E_BI = 500_000, 300_000, 400_000

NT = 16
NC = 2
BATCH = 2048
DBATCH = 1024
ZR = 512

LEN_IL, LEN_BL, LEN_BI = 60032, 40064, 20096


def _ceil_to(x, m):
    return (x + m - 1) // m * m


E_IL_PAD = _ceil_to(2 * E_UI, NT * BATCH)
E_BL_PAD = _ceil_to(2 * E_UB, NT * BATCH)
E_BI_PAD = _ceil_to(E_BI, NT * BATCH)

DEG_E = 2 * E_UI + 2 * E_UB + E_BI
DEG_EPAD = _ceil_to(DEG_E, 3 * NC * NT * DBATCH)
OFF_BL = LEN_IL
OFF_BI = LEN_IL + LEN_BL
N_DEG = LEN_IL + LEN_BL + LEN_BI
DUMMY_DEG = N_IL


def _mesh():
    return plsc.VectorSubcoreMesh(core_axis_name="c", subcore_axis_name="s")


@functools.cache
def _deg_kernel():
    half = DEG_EPAD // NC
    chunk = half // NT
    nb = chunk // DBATCH
    nzz = N_DEG // NT
    n1, n2, n3 = LEN_IL // NT, LEN_BL // NT, LEN_BI // NT

    nb3 = nb // 3

    def body(didx, o_il, o_bl, o_bi, idxv0, idxv1, idxv2, onesv, zbuf, acc,
             isem0, isem1, isem2, ssem0, ssem1, ssem2):
        c = lax.axis_index("c")
        s = lax.axis_index("s")
        idxv = (idxv0, idxv1, idxv2)
        isem = (isem0, isem1, isem2)
        ssem = (ssem0, ssem1, ssem2)
        ones16 = jnp.ones((16,), jnp.float32)
        zero16 = jnp.zeros((16,), jnp.float32)

        def fill(i, _):
            zbuf[pl.ds(i * 16, 16)] = zero16
            return 0

        lax.fori_loop(0, 2048 // 16, fill, 0)

        def fill1(i, _):
            onesv[pl.ds(i * 16, 16)] = ones16
            return 0

        lax.fori_loop(0, DBATCH // 16, fill1, 0)

        off = 0
        while off < nzz:
            sz = min(2048, nzz - off)
            pltpu.sync_copy(zbuf.at[pl.ds(0, sz)], acc.at[pl.ds(s * nzz + off, sz)])
            off += sz
        plsc.subcore_barrier()

        base = c * half + s * chunk

        def idx_start(b, k):
            pltpu.async_copy(didx.at[pl.ds(base + b * DBATCH, DBATCH)], idxv[k], isem[k])

        def idx_wait(b, k):
            pltpu.make_async_copy(
                didx.at[pl.ds(base + b * DBATCH, DBATCH)], idxv[k], isem[k]).wait()

        def scat_start(k):
            pltpu.async_copy(onesv, acc.at[idxv[k]], ssem[k], add=True)

        def scat_wait(k):
            pltpu.make_async_copy(onesv, acc.at[idxv[k]], ssem[k]).wait()

        def slot(b, k, do_c):
            k2 = (k + 1) % 3
            idx_wait(b, k)
            scat_start(k)
            if do_c:
                scat_wait(k2)
            idx_start(jnp.minimum(b + 1, nb - 1), k2)

        idx_start(0, 0)
        slot(0, 0, False)
        slot(1, 1, False)
        slot(2, 2, True)

        def outer(j, _):
            b = 3 * j
            slot(b, 0, True)
            slot(b + 1, 1, True)
            slot(b + 2, 2, True)
            return 0

        lax.fori_loop(1, nb3, outer, 0)
        scat_wait(1)
        scat_wait(2)
        idx_wait(nb - 1, 0)
        plsc.subcore_barrier()

        def drain(a_off, out, o_off, total):
            done = 0
            while done < total:
                sz = min(2048, total - done)
                pltpu.sync_copy(acc.at[pl.ds(a_off + done, sz)], zbuf.at[pl.ds(0, sz)])
                pltpu.sync_copy(zbuf.at[pl.ds(0, sz)], out.at[pl.ds(o_off + done, sz)])
                done += sz

        drain(s * n1, o_il, c * LEN_IL + s * n1, n1)
        drain(OFF_BL + s * n2, o_bl, c * LEN_BL + s * n2, n2)
        drain(OFF_BI + s * n3, o_bi, c * LEN_BI + s * n3, n3)

    return pl.kernel(
        body,
        out_type=[
            jax.ShapeDtypeStruct((NC * LEN_IL,), jnp.float32),
            jax.ShapeDtypeStruct((NC * LEN_BL,), jnp.float32),
            jax.ShapeDtypeStruct((NC * LEN_BI,), jnp.float32),
        ],
        mesh=_mesh(),
        compiler_params=pltpu.CompilerParams(use_tc_tiling_on_sc=False),
        scratch_types=[
            pltpu.VMEM((DBATCH,), jnp.int32),
            pltpu.VMEM((DBATCH,), jnp.int32),
            pltpu.VMEM((DBATCH,), jnp.int32),
            pltpu.VMEM((DBATCH,), jnp.float32),
            pltpu.VMEM((2048,), jnp.float32),
            pltpu.VMEM_SHARED((N_DEG,), jnp.float32),
            pltpu.SemaphoreType.DMA,
            pltpu.SemaphoreType.DMA,
            pltpu.SemaphoreType.DMA,
            pltpu.SemaphoreType.DMA,
            pltpu.SemaphoreType.DMA,
            pltpu.SemaphoreType.DMA,
        ],
    )


@functools.cache
def _spmm(n_pad, e_pad, pass_off):
    chunk = e_pad // NT
    nb = chunk // BATCH
    nro = n_pad // NT

    def body(table, cols4, rows, out, colv0, rowv0, gath0, ztile, acc, gsem0):
        c = lax.axis_index("c")
        s = lax.axis_index("s")
        zv = jnp.zeros((16,), jnp.float32)

        def zfill(i, _):
            ztile[i, pl.ds(0, 16)] = zv
            return 0

        lax.fori_loop(0, ZR, zfill, 0)
        off = 0
        while off < nro:
            sz = min(ZR, nro - off)
            pltpu.sync_copy(ztile.at[pl.ds(0, sz)], acc.at[pl.ds(s * nro + off, sz)])
            off += sz
        plsc.subcore_barrier()

        base = s * chunk

        def step(b, _):
            eb = base + b * BATCH
            pltpu.sync_copy(cols4.at[pass_off + c, pl.ds(eb, BATCH)], colv0)
            pltpu.sync_copy(rows.at[pl.ds(eb, BATCH)], rowv0)
            pltpu.async_copy(table.at[colv0], gath0, gsem0).wait()
            pltpu.sync_copy(gath0, acc.at[rowv0], add=True)
            return 0

        lax.fori_loop(0, nb, step, 0)
        plsc.subcore_barrier()
        done = 0
        while done < nro:
            sz = min(ZR, nro - done)
            pltpu.sync_copy(acc.at[pl.ds(s * nro + done, sz)], ztile.at[pl.ds(0, sz)])
            pltpu.sync_copy(ztile.at[pl.ds(0, sz)], out.at[c, pl.ds(s * nro + done, sz)])
            done += sz

    return pl.kernel(
        body,
        out_type=jax.ShapeDtypeStruct((NC, n_pad, W), jnp.float32),
        mesh=_mesh(),
        compiler_params=pltpu.CompilerParams(use_tc_tiling_on_sc=False),
        scratch_types=[
            pltpu.VMEM((BATCH,), jnp.int32),
            pltpu.VMEM((BATCH,), jnp.int32),
            pltpu.VMEM((BATCH, W), jnp.float32),
            pltpu.VMEM((ZR, W), jnp.float32),
            pltpu.VMEM_SHARED((n_pad, W), jnp.float32),
            pltpu.SemaphoreType.DMA,
        ],
    )


BR = 128


def _dinv_of(dref):
    d = dref[0] + dref[1]
    return 1.0 / (jnp.sqrt(d) + 1e-8)


def _quarters(oref, x):
    for q in range(4):
        oref[q] = x[:, q * W:(q + 1) * W]


def _tc_prescale(f0, deg):
    n = f0.shape[0]

    def body(fref, dref, oref):
        g = fref[...] * _dinv_of(dref)
        _quarters(oref, g)

    return pl.pallas_call(
        body,
        grid=(n // BR,),
        in_specs=[
            pl.BlockSpec((BR, D), lambda i: (i, 0)),
            pl.BlockSpec((2, BR, 1), lambda i: (0, i, 0)),
        ],
        out_specs=pl.BlockSpec((4, BR, W), lambda i: (0, i, 0)),
        out_shape=jax.ShapeDtypeStruct((4, n, W), jnp.float32),
    )(f0, deg)


def _tc_post(sa, sb, deg, prev, scale, want_g, want_ostk):
    n = prev.shape[0]
    out_shape = [jax.ShapeDtypeStruct((n, D), jnp.float32)]
    out_specs = [pl.BlockSpec((BR, D), lambda i: (i, 0))]
    if want_g or want_ostk:
        out_shape.append(jax.ShapeDtypeStruct((4, n, W), jnp.float32))
        out_specs.append(pl.BlockSpec((4, BR, W), lambda i: (0, i, 0)))

    def body(saref, sbref, dref, pref, oref, *rest):
        dinv = _dinv_of(dref)
        s2 = jnp.concatenate([saref[0], saref[1], sbref[0], sbref[1]], axis=1)
        f = s2 * dinv * scale
        nrm = jnp.sqrt(jnp.sum(f * f, axis=1, keepdims=True))
        a = f / (nrm + 1e-12)
        p = pref[...] + a
        oref[...] = p
        if want_g:
            _quarters(rest[0], f * dinv)
        elif want_ostk:
            _quarters(rest[0], p)

    return pl.pallas_call(
        body,
        grid=(n // BR,),
        in_specs=[
            pl.BlockSpec((2, BR, W), lambda i: (0, i, 0)),
            pl.BlockSpec((2, BR, W), lambda i: (0, i, 0)),
            pl.BlockSpec((2, BR, 1), lambda i: (0, i, 0)),
            pl.BlockSpec((BR, D), lambda i: (i, 0)),
        ],
        out_specs=out_specs,
        out_shape=out_shape,
    )(sa, sb, deg, prev)


def _tc_post_bi(sa, sb, deg):
    n = deg.shape[1]

    def body(saref, sbref, dref, oref):
        binv = 1.0 / (dref[0] + dref[1] + 1e-8)
        s2 = jnp.concatenate([saref[0], saref[1], sbref[0], sbref[1]], axis=1)
        oref[...] = s2 * binv

    return pl.pallas_call(
        body,
        grid=(n // BR,),
        in_specs=[
            pl.BlockSpec((2, BR, W), lambda i: (0, i, 0)),
            pl.BlockSpec((2, BR, W), lambda i: (0, i, 0)),
            pl.BlockSpec((2, BR, 1), lambda i: (0, i, 0)),
        ],
        out_specs=pl.BlockSpec((BR, D), lambda i: (i, 0)),
        out_shape=jax.ShapeDtypeStruct((n, D), jnp.float32),
    )(sa, sb, deg)


def _pad_i32(x, length, value):
    return jnp.concatenate([x, jnp.full((length - x.shape[0],), value, jnp.int32)])


def _spmm4(table4, cols4, rows, n_pad):
    tbl = table4.reshape(-1, W)
    e_pad = rows.shape[0]
    sa = _spmm(n_pad, e_pad, 0)(tbl, cols4, rows)
    sb = _spmm(n_pad, e_pad, 2)(tbl, cols4, rows)
    return sa, sb


def _propagate(f0, rows, cols4, deg, n_pad):
    g0 = _tc_prescale(f0, deg)
    sa1, sb1 = _spmm4(g0, cols4, rows, n_pad)
    p1, g1 = _tc_post(sa1, sb1, deg, f0, 0.5, True, False)
    sa2, sb2 = _spmm4(g1, cols4, rows, n_pad)
    return _tc_post(sa2, sb2, deg, p1, 1.0 / 3.0, False, True)


def _cols4_of(cols, n_pad):
    return jnp.stack([cols, cols + n_pad, cols + 2 * n_pad, cols + 3 * n_pad])


def kernel(users_feature, items_feature, bundles_feature, ui_edges, ub_edges, bi_edges):
    ur, uc = ui_edges[0], ui_edges[1]
    vr, vc = ub_edges[0], ub_edges[1]
    br_, bc_ = bi_edges[0], bi_edges[1]

    deg_idx = jnp.concatenate([
        ur, uc + NU,
        vr + OFF_BL, vc + (NU + OFF_BL),
        br_ + OFF_BI,
        jnp.full((DEG_EPAD - DEG_E,), DUMMY_DEG, jnp.int32),
    ])
    d_il, d_bl, d_bi = _deg_kernel()(deg_idx)
    deg_il = d_il.reshape(NC, LEN_IL, 1)
    deg_bl = d_bl.reshape(NC, LEN_BL, 1)
    deg_bi = d_bi.reshape(NC, LEN_BI, 1)

    zpad_il = jnp.zeros((LEN_IL - N_IL, D), jnp.float32)
    zpad_bl = jnp.zeros((LEN_BL - N_BL, D), jnp.float32)

    f0_il = jnp.concatenate([users_feature, items_feature, zpad_il], axis=0)
    rows_il = _pad_i32(jnp.concatenate([ur, uc + NU]), E_IL_PAD, N_IL)
    cols_il = _pad_i32(jnp.concatenate([uc + NU, ur]), E_IL_PAD, 0)
    il_out, il_stk = _propagate(f0_il, rows_il, _cols4_of(cols_il, LEN_IL), deg_il, LEN_IL)

    f0_bl = jnp.concatenate([users_feature, bundles_feature, zpad_bl], axis=0)
    rows_bl = _pad_i32(jnp.concatenate([vr, vc + NU]), E_BL_PAD, N_BL)
    cols_bl = _pad_i32(jnp.concatenate([vc + NU, vr]), E_BL_PAD, 0)
    bl_out, _ = _propagate(f0_bl, rows_bl, _cols4_of(cols_bl, LEN_BL), deg_bl, LEN_BL)

    rows_bi = _pad_i32(br_, E_BI_PAD, NB)
    cols_bi = _pad_i32(bc_ + NU, E_BI_PAD, 0)
    sa, sb = _spmm4(il_stk, _cols4_of(cols_bi, LEN_IL), rows_bi, LEN_BI)
    il_b = _tc_post_bi(sa, sb, deg_bi)

    users_rep = jnp.concatenate([il_out[:NU], bl_out[:NU]], axis=1)
    bundles_rep = jnp.concatenate([il_b[:NB], bl_out[NU:N_BL]], axis=1)
    return users_rep, bundles_rep


# speedup vs baseline: 1.1608x; 1.0522x over previous
import functools

import jax
import jax.numpy as jnp
from jax import lax
from jax.experimental import pallas as pl
from jax.experimental.pallas import tpu as pltpu
from jax.experimental.pallas import tpu_sc as plsc

NU, NI, NB = 20000, 40000, 20000
D, H, W = 64, 32, 16
N_IL = NU + NI
N_BL = NU + NB
E_UI, E_UB, E_BI = 500_000, 300_000, 400_000

NT = 16
NC = 2
BATCH = 1024
DBATCH = 1024
ZR = 512

LEN_IL, LEN_BL, LEN_BI = 60032, 40064, 20096


def _ceil_to(x, m):
    return (x + m - 1) // m * m


E_IL_PAD = _ceil_to(2 * E_UI, NT * BATCH)
E_BL_PAD = _ceil_to(2 * E_UB, NT * BATCH)
E_BI_PAD = _ceil_to(E_BI, NT * BATCH)

DEG_E = 2 * E_UI + 2 * E_UB + E_BI
DEG_EPAD = _ceil_to(DEG_E, 3 * NC * NT * DBATCH)
OFF_BL = LEN_IL
OFF_BI = LEN_IL + LEN_BL
N_DEG = LEN_IL + LEN_BL + LEN_BI
DUMMY_DEG = N_IL


def _mesh():
    return plsc.VectorSubcoreMesh(core_axis_name="c", subcore_axis_name="s")


@functools.cache
def _deg_kernel():
    half = DEG_EPAD // NC
    chunk = half // NT
    nb = chunk // DBATCH
    nzz = N_DEG // NT
    n1, n2, n3 = LEN_IL // NT, LEN_BL // NT, LEN_BI // NT

    nb3 = nb // 3

    def body(didx, o_il, o_bl, o_bi, idxv0, idxv1, idxv2, onesv, zbuf, acc,
             isem0, isem1, isem2, ssem0, ssem1, ssem2):
        c = lax.axis_index("c")
        s = lax.axis_index("s")
        idxv = (idxv0, idxv1, idxv2)
        isem = (isem0, isem1, isem2)
        ssem = (ssem0, ssem1, ssem2)
        ones16 = jnp.ones((16,), jnp.float32)
        zero16 = jnp.zeros((16,), jnp.float32)

        def fill(i, _):
            zbuf[pl.ds(i * 16, 16)] = zero16
            return 0

        lax.fori_loop(0, 2048 // 16, fill, 0)

        def fill1(i, _):
            onesv[pl.ds(i * 16, 16)] = ones16
            return 0

        lax.fori_loop(0, DBATCH // 16, fill1, 0)

        off = 0
        while off < nzz:
            sz = min(2048, nzz - off)
            pltpu.sync_copy(zbuf.at[pl.ds(0, sz)], acc.at[pl.ds(s * nzz + off, sz)])
            off += sz
        plsc.subcore_barrier()

        base = c * half + s * chunk

        def idx_start(b, k):
            pltpu.async_copy(didx.at[pl.ds(base + b * DBATCH, DBATCH)], idxv[k], isem[k])

        def idx_wait(b, k):
            pltpu.make_async_copy(
                didx.at[pl.ds(base + b * DBATCH, DBATCH)], idxv[k], isem[k]).wait()

        def scat_start(k):
            pltpu.async_copy(onesv, acc.at[idxv[k]], ssem[k], add=True)

        def scat_wait(k):
            pltpu.make_async_copy(onesv, acc.at[idxv[k]], ssem[k]).wait()

        def slot(b, k, do_c):
            k2 = (k + 1) % 3
            idx_wait(b, k)
            scat_start(k)
            if do_c:
                scat_wait(k2)
            idx_start(jnp.minimum(b + 1, nb - 1), k2)

        idx_start(0, 0)
        slot(0, 0, False)
        slot(1, 1, False)
        slot(2, 2, True)

        def outer(j, _):
            b = 3 * j
            slot(b, 0, True)
            slot(b + 1, 1, True)
            slot(b + 2, 2, True)
            return 0

        lax.fori_loop(1, nb3, outer, 0)
        scat_wait(1)
        scat_wait(2)
        idx_wait(nb - 1, 0)
        plsc.subcore_barrier()

        def drain(a_off, out, o_off, total):
            done = 0
            while done < total:
                sz = min(2048, total - done)
                pltpu.sync_copy(acc.at[pl.ds(a_off + done, sz)], zbuf.at[pl.ds(0, sz)])
                pltpu.sync_copy(zbuf.at[pl.ds(0, sz)], out.at[pl.ds(o_off + done, sz)])
                done += sz

        drain(s * n1, o_il, c * LEN_IL + s * n1, n1)
        drain(OFF_BL + s * n2, o_bl, c * LEN_BL + s * n2, n2)
        drain(OFF_BI + s * n3, o_bi, c * LEN_BI + s * n3, n3)

    return pl.kernel(
        body,
        out_type=[
            jax.ShapeDtypeStruct((NC * LEN_IL,), jnp.float32),
            jax.ShapeDtypeStruct((NC * LEN_BL,), jnp.float32),
            jax.ShapeDtypeStruct((NC * LEN_BI,), jnp.float32),
        ],
        mesh=_mesh(),
        compiler_params=pltpu.CompilerParams(use_tc_tiling_on_sc=False),
        scratch_types=[
            pltpu.VMEM((DBATCH,), jnp.int32),
            pltpu.VMEM((DBATCH,), jnp.int32),
            pltpu.VMEM((DBATCH,), jnp.int32),
            pltpu.VMEM((DBATCH,), jnp.float32),
            pltpu.VMEM((2048,), jnp.float32),
            pltpu.VMEM_SHARED((N_DEG,), jnp.float32),
            pltpu.SemaphoreType.DMA,
            pltpu.SemaphoreType.DMA,
            pltpu.SemaphoreType.DMA,
            pltpu.SemaphoreType.DMA,
            pltpu.SemaphoreType.DMA,
            pltpu.SemaphoreType.DMA,
        ],
    )


@functools.cache
def _spmm(n_pad, e_pad, pass_off):
    chunk = e_pad // NT
    nb = chunk // BATCH
    nro = n_pad // NT

    def body(table, cols4, rows, out, colv0, rowv0, gath0, ztile, acc, gsem0):
        c = lax.axis_index("c")
        s = lax.axis_index("s")
        zv = jnp.zeros((16,), jnp.float32)

        def zfill(i, _):
            ztile[i, pl.ds(0, 16)] = zv
            return 0

        lax.fori_loop(0, ZR, zfill, 0)
        off = 0
        while off < nro:
            sz = min(ZR, nro - off)
            pltpu.sync_copy(ztile.at[pl.ds(0, sz)], acc.at[pl.ds(s * nro + off, sz)])
            off += sz
        plsc.subcore_barrier()

        base = s * chunk

        def step(b, _):
            eb = base + b * BATCH
            pltpu.sync_copy(cols4.at[pass_off + c, pl.ds(eb, BATCH)], colv0)
            pltpu.sync_copy(rows.at[pl.ds(eb, BATCH)], rowv0)
            pltpu.async_copy(table.at[colv0], gath0, gsem0).wait()
            pltpu.sync_copy(gath0, acc.at[rowv0], add=True)
            return 0

        lax.fori_loop(0, nb, step, 0)
        plsc.subcore_barrier()
        done = 0
        while done < nro:
            sz = min(ZR, nro - done)
            pltpu.sync_copy(acc.at[pl.ds(s * nro + done, sz)], ztile.at[pl.ds(0, sz)])
            pltpu.sync_copy(ztile.at[pl.ds(0, sz)], out.at[c, pl.ds(s * nro + done, sz)])
            done += sz

    return pl.kernel(
        body,
        out_type=jax.ShapeDtypeStruct((NC, n_pad, W), jnp.float32),
        mesh=_mesh(),
        compiler_params=pltpu.CompilerParams(use_tc_tiling_on_sc=False),
        scratch_types=[
            pltpu.VMEM((BATCH,), jnp.int32),
            pltpu.VMEM((BATCH,), jnp.int32),
            pltpu.VMEM((BATCH, W), jnp.float32),
            pltpu.VMEM((ZR, W), jnp.float32),
            pltpu.VMEM_SHARED((n_pad, W), jnp.float32),
            pltpu.SemaphoreType.DMA,
        ],
    )


BR = 128


def _dinv_of(dref):
    d = dref[0] + dref[1]
    return 1.0 / (jnp.sqrt(d) + 1e-8)


def _quarters(oref, x):
    for q in range(4):
        oref[q] = x[:, q * W:(q + 1) * W]


def _tc_prescale(f0, deg):
    n = f0.shape[0]

    def body(fref, dref, oref):
        g = fref[...] * _dinv_of(dref)
        _quarters(oref, g)

    return pl.pallas_call(
        body,
        grid=(n // BR,),
        in_specs=[
            pl.BlockSpec((BR, D), lambda i: (i, 0)),
            pl.BlockSpec((2, BR, 1), lambda i: (0, i, 0)),
        ],
        out_specs=pl.BlockSpec((4, BR, W), lambda i: (0, i, 0)),
        out_shape=jax.ShapeDtypeStruct((4, n, W), jnp.float32),
    )(f0, deg)


def _tc_post(sa, sb, deg, prev, scale, want_g, want_ostk):
    n = prev.shape[0]
    out_shape = [jax.ShapeDtypeStruct((n, D), jnp.float32)]
    out_specs = [pl.BlockSpec((BR, D), lambda i: (i, 0))]
    if want_g or want_ostk:
        out_shape.append(jax.ShapeDtypeStruct((4, n, W), jnp.float32))
        out_specs.append(pl.BlockSpec((4, BR, W), lambda i: (0, i, 0)))

    def body(saref, sbref, dref, pref, oref, *rest):
        dinv = _dinv_of(dref)
        s2 = jnp.concatenate([saref[0], saref[1], sbref[0], sbref[1]], axis=1)
        f = s2 * dinv * scale
        nrm = jnp.sqrt(jnp.sum(f * f, axis=1, keepdims=True))
        a = f / (nrm + 1e-12)
        p = pref[...] + a
        oref[...] = p
        if want_g:
            _quarters(rest[0], f * dinv)
        elif want_ostk:
            _quarters(rest[0], p)

    return pl.pallas_call(
        body,
        grid=(n // BR,),
        in_specs=[
            pl.BlockSpec((2, BR, W), lambda i: (0, i, 0)),
            pl.BlockSpec((2, BR, W), lambda i: (0, i, 0)),
            pl.BlockSpec((2, BR, 1), lambda i: (0, i, 0)),
            pl.BlockSpec((BR, D), lambda i: (i, 0)),
        ],
        out_specs=out_specs,
        out_shape=out_shape,
    )(sa, sb, deg, prev)


def _tc_post_bi(sa, sb, deg):
    n = deg.shape[1]

    def body(saref, sbref, dref, oref):
        binv = 1.0 / (dref[0] + dref[1] + 1e-8)
        s2 = jnp.concatenate([saref[0], saref[1], sbref[0], sbref[1]], axis=1)
        oref[...] = s2 * binv

    return pl.pallas_call(
        body,
        grid=(n // BR,),
        in_specs=[
            pl.BlockSpec((2, BR, W), lambda i: (0, i, 0)),
            pl.BlockSpec((2, BR, W), lambda i: (0, i, 0)),
            pl.BlockSpec((2, BR, 1), lambda i: (0, i, 0)),
        ],
        out_specs=pl.BlockSpec((BR, D), lambda i: (i, 0)),
        out_shape=jax.ShapeDtypeStruct((n, D), jnp.float32),
    )(sa, sb, deg)


def _pad_i32(x, length, value):
    return jnp.concatenate([x, jnp.full((length - x.shape[0],), value, jnp.int32)])


def _spmm4(table4, cols4, rows, n_pad):
    tbl = table4.reshape(-1, W)
    e_pad = rows.shape[0]
    sa = _spmm(n_pad, e_pad, 0)(tbl, cols4, rows)
    sb = _spmm(n_pad, e_pad, 2)(tbl, cols4, rows)
    return sa, sb


def _propagate(f0, rows, cols4, deg, n_pad):
    g0 = _tc_prescale(f0, deg)
    sa1, sb1 = _spmm4(g0, cols4, rows, n_pad)
    p1, g1 = _tc_post(sa1, sb1, deg, f0, 0.5, True, False)
    sa2, sb2 = _spmm4(g1, cols4, rows, n_pad)
    return _tc_post(sa2, sb2, deg, p1, 1.0 / 3.0, False, True)


def _cols4_of(cols, n_pad):
    return jnp.stack([cols, cols + n_pad, cols + 2 * n_pad, cols + 3 * n_pad])


def kernel(users_feature, items_feature, bundles_feature, ui_edges, ub_edges, bi_edges):
    ur, uc = ui_edges[0], ui_edges[1]
    vr, vc = ub_edges[0], ub_edges[1]
    br_, bc_ = bi_edges[0], bi_edges[1]

    deg_idx = jnp.concatenate([
        ur, uc + NU,
        vr + OFF_BL, vc + (NU + OFF_BL),
        br_ + OFF_BI,
        jnp.full((DEG_EPAD - DEG_E,), DUMMY_DEG, jnp.int32),
    ])
    d_il, d_bl, d_bi = _deg_kernel()(deg_idx)
    deg_il = d_il.reshape(NC, LEN_IL, 1)
    deg_bl = d_bl.reshape(NC, LEN_BL, 1)
    deg_bi = d_bi.reshape(NC, LEN_BI, 1)

    zpad_il = jnp.zeros((LEN_IL - N_IL, D), jnp.float32)
    zpad_bl = jnp.zeros((LEN_BL - N_BL, D), jnp.float32)

    f0_il = jnp.concatenate([users_feature, items_feature, zpad_il], axis=0)
    rows_il = _pad_i32(jnp.concatenate([ur, uc + NU]), E_IL_PAD, N_IL)
    cols_il = _pad_i32(jnp.concatenate([uc + NU, ur]), E_IL_PAD, 0)
    il_out, il_stk = _propagate(f0_il, rows_il, _cols4_of(cols_il, LEN_IL), deg_il, LEN_IL)

    f0_bl = jnp.concatenate([users_feature, bundles_feature, zpad_bl], axis=0)
    rows_bl = _pad_i32(jnp.concatenate([vr, vc + NU]), E_BL_PAD, N_BL)
    cols_bl = _pad_i32(jnp.concatenate([vc + NU, vr]), E_BL_PAD, 0)
    bl_out, _ = _propagate(f0_bl, rows_bl, _cols4_of(cols_bl, LEN_BL), deg_bl, LEN_BL)

    rows_bi = _pad_i32(br_, E_BI_PAD, NB)
    cols_bi = _pad_i32(bc_ + NU, E_BI_PAD, 0)
    sa, sb = _spmm4(il_stk, _cols4_of(cols_bi, LEN_IL), rows_bi, LEN_BI)
    il_b = _tc_post_bi(sa, sb, deg_bi)

    users_rep = jnp.concatenate([il_out[:NU], bl_out[:NU]], axis=1)
    bundles_rep = jnp.concatenate([il_b[:NB], bl_out[NU:N_BL]], axis=1)
    return users_rep, bundles_rep


# speedup vs baseline: 1.1999x; 1.0337x over previous
import functools

import jax
import jax.numpy as jnp
from jax import lax
from jax.experimental import pallas as pl
from jax.experimental.pallas import tpu as pltpu
from jax.experimental.pallas import tpu_sc as plsc

NU, NI, NB = 20000, 40000, 20000
D, H, W = 64, 32, 16
N_IL = NU + NI
N_BL = NU + NB
E_UI, E_UB, E_BI = 500_000, 300_000, 400_000

NT = 16
NC = 2
BATCH = 1024
DBATCH = 1024
ZR = 512

LEN_IL, LEN_BL, LEN_BI = 60032, 40064, 20096


def _ceil_to(x, m):
    return (x + m - 1) // m * m


E_IL_PAD = _ceil_to(2 * E_UI, NT * BATCH)
E_BL_PAD = _ceil_to(2 * E_UB, NT * BATCH)
E_BI_PAD = _ceil_to(E_BI, NT * BATCH)

DEG_E = 2 * E_UI + 2 * E_UB + E_BI
DEG_EPAD = _ceil_to(DEG_E, 3 * NC * NT * DBATCH)
OFF_BL = LEN_IL
OFF_BI = LEN_IL + LEN_BL
N_DEG = LEN_IL + LEN_BL + LEN_BI
DUMMY_DEG = N_IL


def _mesh():
    return plsc.VectorSubcoreMesh(core_axis_name="c", subcore_axis_name="s")


@functools.cache
def _deg_kernel():
    half = DEG_EPAD // NC
    chunk = half // NT
    nb = chunk // DBATCH
    nzz = N_DEG // NT
    n1, n2, n3 = LEN_IL // NT, LEN_BL // NT, LEN_BI // NT

    nb3 = nb // 3

    def body(didx, o_il, o_bl, o_bi, idxv0, idxv1, idxv2, onesv, zbuf, acc,
             isem0, isem1, isem2, ssem0, ssem1, ssem2):
        c = lax.axis_index("c")
        s = lax.axis_index("s")
        idxv = (idxv0, idxv1, idxv2)
        isem = (isem0, isem1, isem2)
        ssem = (ssem0, ssem1, ssem2)
        ones16 = jnp.ones((16,), jnp.float32)
        zero16 = jnp.zeros((16,), jnp.float32)

        def fill(i, _):
            zbuf[pl.ds(i * 16, 16)] = zero16
            return 0

        lax.fori_loop(0, 2048 // 16, fill, 0)

        def fill1(i, _):
            onesv[pl.ds(i * 16, 16)] = ones16
            return 0

        lax.fori_loop(0, DBATCH // 16, fill1, 0)

        off = 0
        while off < nzz:
            sz = min(2048, nzz - off)
            pltpu.sync_copy(zbuf.at[pl.ds(0, sz)], acc.at[pl.ds(s * nzz + off, sz)])
            off += sz
        plsc.subcore_barrier()

        base = c * half + s * chunk

        def idx_start(b, k):
            pltpu.async_copy(didx.at[pl.ds(base + b * DBATCH, DBATCH)], idxv[k], isem[k])

        def idx_wait(b, k):
            pltpu.make_async_copy(
                didx.at[pl.ds(base + b * DBATCH, DBATCH)], idxv[k], isem[k]).wait()

        def scat_start(k):
            pltpu.async_copy(onesv, acc.at[idxv[k]], ssem[k], add=True)

        def scat_wait(k):
            pltpu.make_async_copy(onesv, acc.at[idxv[k]], ssem[k]).wait()

        def slot(b, k, do_c):
            k2 = (k + 1) % 3
            idx_wait(b, k)
            scat_start(k)
            if do_c:
                scat_wait(k2)
            idx_start(jnp.minimum(b + 1, nb - 1), k2)

        idx_start(0, 0)
        slot(0, 0, False)
        slot(1, 1, False)
        slot(2, 2, True)

        def outer(j, _):
            b = 3 * j
            slot(b, 0, True)
            slot(b + 1, 1, True)
            slot(b + 2, 2, True)
            return 0

        lax.fori_loop(1, nb3, outer, 0)
        scat_wait(1)
        scat_wait(2)
        idx_wait(nb - 1, 0)
        plsc.subcore_barrier()

        def drain(a_off, out, o_off, total):
            done = 0
            while done < total:
                sz = min(2048, total - done)
                pltpu.sync_copy(acc.at[pl.ds(a_off + done, sz)], zbuf.at[pl.ds(0, sz)])
                pltpu.sync_copy(zbuf.at[pl.ds(0, sz)], out.at[pl.ds(o_off + done, sz)])
                done += sz

        drain(s * n1, o_il, c * LEN_IL + s * n1, n1)
        drain(OFF_BL + s * n2, o_bl, c * LEN_BL + s * n2, n2)
        drain(OFF_BI + s * n3, o_bi, c * LEN_BI + s * n3, n3)

    return pl.kernel(
        body,
        out_type=[
            jax.ShapeDtypeStruct((NC * LEN_IL,), jnp.float32),
            jax.ShapeDtypeStruct((NC * LEN_BL,), jnp.float32),
            jax.ShapeDtypeStruct((NC * LEN_BI,), jnp.float32),
        ],
        mesh=_mesh(),
        compiler_params=pltpu.CompilerParams(use_tc_tiling_on_sc=False),
        scratch_types=[
            pltpu.VMEM((DBATCH,), jnp.int32),
            pltpu.VMEM((DBATCH,), jnp.int32),
            pltpu.VMEM((DBATCH,), jnp.int32),
            pltpu.VMEM((DBATCH,), jnp.float32),
            pltpu.VMEM((2048,), jnp.float32),
            pltpu.VMEM_SHARED((N_DEG,), jnp.float32),
            pltpu.SemaphoreType.DMA,
            pltpu.SemaphoreType.DMA,
            pltpu.SemaphoreType.DMA,
            pltpu.SemaphoreType.DMA,
            pltpu.SemaphoreType.DMA,
            pltpu.SemaphoreType.DMA,
        ],
    )


@functools.cache
def _spmm(n_pad, e_pad, pass_off):
    chunk = e_pad // NT
    nb = chunk // BATCH
    nro = n_pad // NT

    def body(table, cols4, rows, out, colv0, rowv0, colv1, rowv1,
             gath0, gath1, ztile, acc, gsem0, gsem1, ssem0, ssem1):
        c = lax.axis_index("c")
        s = lax.axis_index("s")
        colv = (colv0, colv1)
        rowv = (rowv0, rowv1)
        gath = (gath0, gath1)
        gsem = (gsem0, gsem1)
        ssem = (ssem0, ssem1)
        zv = jnp.zeros((16,), jnp.float32)

        def zfill(i, _):
            ztile[i, pl.ds(0, 16)] = zv
            return 0

        lax.fori_loop(0, ZR, zfill, 0)
        off = 0
        while off < nro:
            sz = min(ZR, nro - off)
            pltpu.sync_copy(ztile.at[pl.ds(0, sz)], acc.at[pl.ds(s * nro + off, sz)])
            off += sz
        plsc.subcore_barrier()

        base = s * chunk

        def idx_load(b, k):
            eb = base + b * BATCH
            pltpu.sync_copy(cols4.at[pass_off + c, pl.ds(eb, BATCH)], colv[k])
            pltpu.sync_copy(rows.at[pl.ds(eb, BATCH)], rowv[k])

        def slot(b, k, do_w):
            pltpu.make_async_copy(table.at[colv[k]], gath[k], gsem[k]).wait()
            pltpu.async_copy(gath[k], acc.at[rowv[k]], ssem[k], add=True)
            if do_w:
                pltpu.make_async_copy(gath[1 - k], acc.at[rowv[1 - k]], ssem[1 - k]).wait()
            idx_load(jnp.minimum(b + 1, nb - 1), 1 - k)
            pltpu.async_copy(table.at[colv[1 - k]], gath[1 - k], gsem[1 - k])

        idx_load(0, 0)
        pltpu.async_copy(table.at[colv0], gath0, gsem0)
        slot(0, 0, False)

        def outer(j, _):
            b = 2 * j
            slot(b + 1, 1, True)
            slot(b + 2, 0, True)
            return 0

        lax.fori_loop(0, (nb - 2) // 2, outer, 0)
        slot(nb - 1, 1, True)
        pltpu.make_async_copy(table.at[colv0], gath0, gsem0).wait()
        pltpu.make_async_copy(gath1, acc.at[rowv1], ssem1).wait()
        plsc.subcore_barrier()
        done = 0
        while done < nro:
            sz = min(ZR, nro - done)
            pltpu.sync_copy(acc.at[pl.ds(s * nro + done, sz)], ztile.at[pl.ds(0, sz)])
            pltpu.sync_copy(ztile.at[pl.ds(0, sz)], out.at[c, pl.ds(s * nro + done, sz)])
            done += sz

    return pl.kernel(
        body,
        out_type=jax.ShapeDtypeStruct((NC, n_pad, W), jnp.float32),
        mesh=_mesh(),
        compiler_params=pltpu.CompilerParams(use_tc_tiling_on_sc=False),
        scratch_types=[
            pltpu.VMEM((BATCH,), jnp.int32),
            pltpu.VMEM((BATCH,), jnp.int32),
            pltpu.VMEM((BATCH,), jnp.int32),
            pltpu.VMEM((BATCH,), jnp.int32),
            pltpu.VMEM((BATCH, W), jnp.float32),
            pltpu.VMEM((BATCH, W), jnp.float32),
            pltpu.VMEM((ZR, W), jnp.float32),
            pltpu.VMEM_SHARED((n_pad, W), jnp.float32),
            pltpu.SemaphoreType.DMA,
            pltpu.SemaphoreType.DMA,
            pltpu.SemaphoreType.DMA,
            pltpu.SemaphoreType.DMA,
        ],
    )


BR = 128


def _dinv_of(dref):
    d = dref[0] + dref[1]
    return 1.0 / (jnp.sqrt(d) + 1e-8)


def _quarters(oref, x):
    for q in range(4):
        oref[q] = x[:, q * W:(q + 1) * W]


def _tc_prescale(f0, deg):
    n = f0.shape[0]

    def body(fref, dref, oref):
        g = fref[...] * _dinv_of(dref)
        _quarters(oref, g)

    return pl.pallas_call(
        body,
        grid=(n // BR,),
        in_specs=[
            pl.BlockSpec((BR, D), lambda i: (i, 0)),
            pl.BlockSpec((2, BR, 1), lambda i: (0, i, 0)),
        ],
        out_specs=pl.BlockSpec((4, BR, W), lambda i: (0, i, 0)),
        out_shape=jax.ShapeDtypeStruct((4, n, W), jnp.float32),
    )(f0, deg)


def _tc_post(sa, sb, deg, prev, scale, want_g, want_ostk):
    n = prev.shape[0]
    out_shape = [jax.ShapeDtypeStruct((n, D), jnp.float32)]
    out_specs = [pl.BlockSpec((BR, D), lambda i: (i, 0))]
    if want_g or want_ostk:
        out_shape.append(jax.ShapeDtypeStruct((4, n, W), jnp.float32))
        out_specs.append(pl.BlockSpec((4, BR, W), lambda i: (0, i, 0)))

    def body(saref, sbref, dref, pref, oref, *rest):
        dinv = _dinv_of(dref)
        s2 = jnp.concatenate([saref[0], saref[1], sbref[0], sbref[1]], axis=1)
        f = s2 * dinv * scale
        nrm = jnp.sqrt(jnp.sum(f * f, axis=1, keepdims=True))
        a = f / (nrm + 1e-12)
        p = pref[...] + a
        oref[...] = p
        if want_g:
            _quarters(rest[0], f * dinv)
        elif want_ostk:
            _quarters(rest[0], p)

    return pl.pallas_call(
        body,
        grid=(n // BR,),
        in_specs=[
            pl.BlockSpec((2, BR, W), lambda i: (0, i, 0)),
            pl.BlockSpec((2, BR, W), lambda i: (0, i, 0)),
            pl.BlockSpec((2, BR, 1), lambda i: (0, i, 0)),
            pl.BlockSpec((BR, D), lambda i: (i, 0)),
        ],
        out_specs=out_specs,
        out_shape=out_shape,
    )(sa, sb, deg, prev)


def _tc_post_bi(sa, sb, deg):
    n = deg.shape[1]

    def body(saref, sbref, dref, oref):
        binv = 1.0 / (dref[0] + dref[1] + 1e-8)
        s2 = jnp.concatenate([saref[0], saref[1], sbref[0], sbref[1]], axis=1)
        oref[...] = s2 * binv

    return pl.pallas_call(
        body,
        grid=(n // BR,),
        in_specs=[
            pl.BlockSpec((2, BR, W), lambda i: (0, i, 0)),
            pl.BlockSpec((2, BR, W), lambda i: (0, i, 0)),
            pl.BlockSpec((2, BR, 1), lambda i: (0, i, 0)),
        ],
        out_specs=pl.BlockSpec((BR, D), lambda i: (i, 0)),
        out_shape=jax.ShapeDtypeStruct((n, D), jnp.float32),
    )(sa, sb, deg)


def _pad_i32(x, length, value):
    return jnp.concatenate([x, jnp.full((length - x.shape[0],), value, jnp.int32)])


def _spmm4(table4, cols4, rows, n_pad):
    tbl = table4.reshape(-1, W)
    e_pad = rows.shape[0]
    sa = _spmm(n_pad, e_pad, 0)(tbl, cols4, rows)
    sb = _spmm(n_pad, e_pad, 2)(tbl, cols4, rows)
    return sa, sb


def _propagate(f0, rows, cols4, deg, n_pad):
    g0 = _tc_prescale(f0, deg)
    sa1, sb1 = _spmm4(g0, cols4, rows, n_pad)
    p1, g1 = _tc_post(sa1, sb1, deg, f0, 0.5, True, False)
    sa2, sb2 = _spmm4(g1, cols4, rows, n_pad)
    return _tc_post(sa2, sb2, deg, p1, 1.0 / 3.0, False, True)


def _cols4_of(cols, n_pad):
    return jnp.stack([cols, cols + n_pad, cols + 2 * n_pad, cols + 3 * n_pad])


def kernel(users_feature, items_feature, bundles_feature, ui_edges, ub_edges, bi_edges):
    ur, uc = ui_edges[0], ui_edges[1]
    vr, vc = ub_edges[0], ub_edges[1]
    br_, bc_ = bi_edges[0], bi_edges[1]

    deg_idx = jnp.concatenate([
        ur, uc + NU,
        vr + OFF_BL, vc + (NU + OFF_BL),
        br_ + OFF_BI,
        jnp.full((DEG_EPAD - DEG_E,), DUMMY_DEG, jnp.int32),
    ])
    d_il, d_bl, d_bi = _deg_kernel()(deg_idx)
    deg_il = d_il.reshape(NC, LEN_IL, 1)
    deg_bl = d_bl.reshape(NC, LEN_BL, 1)
    deg_bi = d_bi.reshape(NC, LEN_BI, 1)

    zpad_il = jnp.zeros((LEN_IL - N_IL, D), jnp.float32)
    zpad_bl = jnp.zeros((LEN_BL - N_BL, D), jnp.float32)

    f0_il = jnp.concatenate([users_feature, items_feature, zpad_il], axis=0)
    rows_il = _pad_i32(jnp.concatenate([ur, uc + NU]), E_IL_PAD, N_IL)
    cols_il = _pad_i32(jnp.concatenate([uc + NU, ur]), E_IL_PAD, 0)
    il_out, il_stk = _propagate(f0_il, rows_il, _cols4_of(cols_il, LEN_IL), deg_il, LEN_IL)

    f0_bl = jnp.concatenate([users_feature, bundles_feature, zpad_bl], axis=0)
    rows_bl = _pad_i32(jnp.concatenate([vr, vc + NU]), E_BL_PAD, N_BL)
    cols_bl = _pad_i32(jnp.concatenate([vc + NU, vr]), E_BL_PAD, 0)
    bl_out, _ = _propagate(f0_bl, rows_bl, _cols4_of(cols_bl, LEN_BL), deg_bl, LEN_BL)

    rows_bi = _pad_i32(br_, E_BI_PAD, NB)
    cols_bi = _pad_i32(bc_ + NU, E_BI_PAD, 0)
    sa, sb = _spmm4(il_stk, _cols4_of(cols_bi, LEN_IL), rows_bi, LEN_BI)
    il_b = _tc_post_bi(sa, sb, deg_bi)

    users_rep = jnp.concatenate([il_out[:NU], bl_out[:NU]], axis=1)
    bundles_rep = jnp.concatenate([il_b[:NB], bl_out[NU:N_BL]], axis=1)
    return users_rep, bundles_rep
